# initial kernel scaffold (unmeasured)
import jax
import jax.numpy as jnp
from jax import lax
from jax.experimental import pallas as pl
from jax.experimental.pallas import tpu as pltpu

M_PER = 1024
N_COLS = 512
M_TOT = 2048


def kernel(x, dest):
    dest2d = dest.reshape(1, M_PER)

    def body(x_ref, dest_ref, out_ref, x_all, dest_all, send_sems, recv_sems):
        my_x = lax.axis_index("x")
        my_y = lax.axis_index("y")
        partner = (my_x, 1 - my_y)

        barrier_sem = pltpu.get_barrier_semaphore()
        pl.semaphore_signal(
            barrier_sem, inc=1,
            device_id=partner, device_id_type=pl.DeviceIdType.MESH,
        )
        pl.semaphore_wait(barrier_sem, 1)

        my_off = my_y * M_PER

        rdma_x = pltpu.make_async_remote_copy(
            src_ref=x_ref,
            dst_ref=x_all.at[pl.ds(my_off, M_PER), :],
            send_sem=send_sems.at[0],
            recv_sem=recv_sems.at[0],
            device_id=partner,
            device_id_type=pl.DeviceIdType.MESH,
        )
        rdma_x.start()
        rdma_d = pltpu.make_async_remote_copy(
            src_ref=dest_ref,
            dst_ref=dest_all.at[:, pl.ds(my_off, M_PER)],
            send_sem=send_sems.at[1],
            recv_sem=recv_sems.at[1],
            device_id=partner,
            device_id_type=pl.DeviceIdType.MESH,
        )
        rdma_d.start()

        x_all[pl.ds(my_off, M_PER), :] = x_ref[:, :]
        dest_all[:, pl.ds(my_off, M_PER)] = dest_ref[:, :]

        rdma_x.wait()
        rdma_d.wait()

        dest_row = dest_all[:, :]
        maskf = (dest_row == my_y).astype(jnp.float32)
        pos = jnp.cumsum(maskf, axis=1) - 1.0
        iota_p = lax.broadcasted_iota(jnp.float32, (M_PER, M_TOT), 0)
        P = jnp.where((iota_p == pos) & (maskf > 0), 1.0, 0.0)
        out_ref[:, :] = jnp.dot(P, x_all[:, :], preferred_element_type=jnp.float32)

    return pl.pallas_call(
        body,
        out_shape=jax.ShapeDtypeStruct((M_PER, N_COLS), jnp.float32),
        in_specs=[
            pl.BlockSpec(memory_space=pltpu.VMEM),
            pl.BlockSpec(memory_space=pltpu.VMEM),
        ],
        out_specs=pl.BlockSpec(memory_space=pltpu.VMEM),
        scratch_shapes=[
            pltpu.VMEM((M_TOT, N_COLS), jnp.float32),
            pltpu.VMEM((1, M_TOT), jnp.int32),
            pltpu.SemaphoreType.DMA((2,)),
            pltpu.SemaphoreType.DMA((2,)),
        ],
        compiler_params=pltpu.CompilerParams(collective_id=0),
    )(x, dest2d)


# baseline (device time: 32825 ns/iter reference)
import jax
import jax.numpy as jnp
from jax import lax
from jax.experimental import pallas as pl
from jax.experimental.pallas import tpu as pltpu

M_PER = 1024
N_COLS = 512
M_TOT = 2048


def kernel(x, dest):
    dest2d = dest.reshape(1, M_PER)

    def body(x_ref, dest_ref, out_ref, x_all, dest_all, send_sems, recv_sems):
        my_x = lax.axis_index("x")
        my_y = lax.axis_index("y")
        partner = (my_x, 1 - my_y)

        barrier_sem = pltpu.get_barrier_semaphore()
        pl.semaphore_signal(
            barrier_sem, inc=1,
            device_id=partner, device_id_type=pl.DeviceIdType.MESH,
        )
        pl.semaphore_wait(barrier_sem, 1)

        my_off = my_y * M_PER

        rdma_x = pltpu.make_async_remote_copy(
            src_ref=x_ref,
            dst_ref=x_all.at[pl.ds(my_off, M_PER), :],
            send_sem=send_sems.at[0],
            recv_sem=recv_sems.at[0],
            device_id=partner,
            device_id_type=pl.DeviceIdType.MESH,
        )
        rdma_x.start()
        rdma_d = pltpu.make_async_remote_copy(
            src_ref=dest_ref,
            dst_ref=dest_all.at[:, pl.ds(my_off, M_PER)],
            send_sem=send_sems.at[1],
            recv_sem=recv_sems.at[1],
            device_id=partner,
            device_id_type=pl.DeviceIdType.MESH,
        )
        rdma_d.start()

        x_all[pl.ds(my_off, M_PER), :] = x_ref[:, :]
        dest_all[:, pl.ds(my_off, M_PER)] = dest_ref[:, :]

        rdma_x.wait()
        rdma_d.wait()

        dest_row = dest_all[:, :]
        maskf = (dest_row == my_y).astype(jnp.float32)
        tri = (
            lax.broadcasted_iota(jnp.int32, (M_TOT, M_TOT), 0)
            <= lax.broadcasted_iota(jnp.int32, (M_TOT, M_TOT), 1)
        ).astype(jnp.float32)
        pos = (
            jnp.dot(maskf, tri, preferred_element_type=jnp.float32) - 1.0
        ).astype(jnp.int32)
        iota_p = lax.broadcasted_iota(jnp.int32, (M_PER, M_TOT), 0)
        P = jnp.where((iota_p == pos) & (maskf > 0), 1.0, 0.0)
        out_ref[:, :] = jnp.dot(P, x_all[:, :], preferred_element_type=jnp.float32)

    return pl.pallas_call(
        body,
        out_shape=jax.ShapeDtypeStruct((M_PER, N_COLS), jnp.float32),
        in_specs=[
            pl.BlockSpec(memory_space=pltpu.VMEM),
            pl.BlockSpec(memory_space=pltpu.VMEM),
        ],
        out_specs=pl.BlockSpec(memory_space=pltpu.VMEM),
        scratch_shapes=[
            pltpu.VMEM((M_TOT, N_COLS), jnp.float32),
            pltpu.VMEM((1, M_TOT), jnp.int32),
            pltpu.SemaphoreType.DMA((2,)),
            pltpu.SemaphoreType.DMA((2,)),
        ],
        compiler_params=pltpu.CompilerParams(collective_id=0),
    )(x, dest2d)
